# Initial kernel scaffold; baseline (speedup 1.0000x reference)
import jax
import jax.numpy as jnp
from jax.experimental import pallas as pl


def kernel(x, edge_index, gate_w, gate_b):
    raise NotImplementedError("write your pallas kernel here")


# speedup vs baseline: 4.4314x; 4.4314x over previous
import functools

import jax
import jax.numpy as jnp
from jax import lax
from jax.experimental import pallas as pl
from jax.experimental.pallas import tpu as pltpu
from jax.experimental.pallas import tpu_sc as plsc

N = 10000
D = 128
E = 320000

NPAD = 10240
NW = 32
EPAD = 327680
PER_W = EPAD // NW
CHUNK = 128
N_CHUNKS = PER_W // CHUNK
ROWS_PER_SUB = NPAD // 16


def _pq_body(x_ref, w_ref, b_ref, p_ref, q_ref):
    xb = x_ref[...]
    w = w_ref[...]
    wa = w[0, :D]
    wb = w[0, D:]
    p_ref[...] = jnp.sum(xb * wa[None, :], axis=1)
    q_ref[...] = jnp.sum(xb * wb[None, :], axis=1) + b_ref[0]


def _node_projections(x_pad, gate_w, gate_b):
    return pl.pallas_call(
        _pq_body,
        out_shape=(
            jax.ShapeDtypeStruct((NPAD,), jnp.float32),
            jax.ShapeDtypeStruct((NPAD,), jnp.float32),
        ),
        in_specs=[
            pl.BlockSpec((NPAD, D), lambda: (0, 0)),
            pl.BlockSpec((1, 2 * D), lambda: (0, 0)),
            pl.BlockSpec(memory_space=pltpu.SMEM),
        ],
    )(x_pad, gate_w, gate_b)


def _lane_splat(vec, j):
    idx = jnp.full((16, 1), j, jnp.int32)
    dn = lax.GatherDimensionNumbers(
        offset_dims=(), collapsed_slice_dims=(0,), start_index_map=(0,))
    return lax.gather(vec, idx, dn, slice_sizes=(1,),
                      mode=lax.GatherScatterMode.PROMISE_IN_BOUNDS)


def _edge_body(x_hbm, src_hbm, dst_hbm, p_hbm, q_hbm, zer_hbm, out_hbm,
               p_v, q_v, src_v, dst_v, att_v, xs_v, out_sh, sem):
    c = lax.axis_index("c")
    s = lax.axis_index("s")
    w = s * 2 + c

    pltpu.sync_copy(zer_hbm, out_sh.at[pl.ds(s * ROWS_PER_SUB, ROWS_PER_SUB)])
    pltpu.sync_copy(p_hbm, p_v)
    pltpu.sync_copy(q_hbm, q_v)
    plsc.subcore_barrier()

    base = w * PER_W

    def chunk_body(k, carry):
        off = base + k * CHUNK
        pltpu.sync_copy(src_hbm.at[pl.ds(off, CHUNK)], src_v)
        pltpu.sync_copy(dst_hbm.at[pl.ds(off, CHUNK)], dst_v)
        cp = pltpu.async_copy(x_hbm.at[src_v], xs_v, sem)
        for g in range(CHUNK // 16):
            sv = src_v[pl.ds(g * 16, 16)]
            dv = dst_v[pl.ds(g * 16, 16)]
            pg = plsc.load_gather(p_v, [sv])
            qg = plsc.load_gather(q_v, [dv])
            t = pg + qg
            att_v[pl.ds(g * 16, 16)] = 1.0 / (1.0 + jnp.exp(-t))
        cp.wait()

        def scale_group(g, carry2):
            ag = att_v[pl.ds(g * 16, 16)]
            for j in range(16):
                sj = _lane_splat(ag, j)
                row = g * 16 + j
                for dcol in range(D // 16):
                    sl = (row, pl.ds(dcol * 16, 16))
                    xs_v[sl] = xs_v[sl] * sj
            return carry2

        lax.fori_loop(0, CHUNK // 16, scale_group, 0)
        pltpu.sync_copy(xs_v, out_sh.at[dst_v], add=True)
        return carry

    lax.fori_loop(0, N_CHUNKS, chunk_body, 0)
    plsc.subcore_barrier()
    r0 = s * ROWS_PER_SUB
    pltpu.sync_copy(out_sh.at[pl.ds(r0, ROWS_PER_SUB)],
                    out_hbm.at[c, pl.ds(r0, ROWS_PER_SUB)])


_edge_kernel = functools.partial(
    pl.kernel,
    out_type=jax.ShapeDtypeStruct((2, NPAD, D), jnp.float32),
    mesh=plsc.VectorSubcoreMesh(core_axis_name="c", subcore_axis_name="s"),
    compiler_params=pltpu.CompilerParams(needs_layout_passes=False),
    scratch_types=[
        pltpu.VMEM((NPAD,), jnp.float32),
        pltpu.VMEM((NPAD,), jnp.float32),
        pltpu.VMEM((CHUNK,), jnp.int32),
        pltpu.VMEM((CHUNK,), jnp.int32),
        pltpu.VMEM((CHUNK,), jnp.float32),
        pltpu.VMEM((CHUNK, D), jnp.float32),
        pltpu.VMEM_SHARED((NPAD, D), jnp.float32),
        pltpu.SemaphoreType.DMA,
    ],
)(_edge_body)


def _combine_body(part_ref, o_ref):
    o_ref[...] = part_ref[0] + part_ref[1]


def _combine(partials):
    return pl.pallas_call(
        _combine_body,
        grid=(5,),
        in_specs=[pl.BlockSpec((2, 2000, D), lambda i: (0, i, 0))],
        out_specs=pl.BlockSpec((2000, D), lambda i: (i, 0)),
        out_shape=jax.ShapeDtypeStruct((N, D), jnp.float32),
    )(partials)


def kernel(x, edge_index, gate_w, gate_b):
    x2d = x.reshape(N, D)
    x_pad = jnp.pad(x2d, ((0, NPAD - N), (0, 0)))
    src = edge_index[0].astype(jnp.int32)
    dst = edge_index[1].astype(jnp.int32)
    src = jnp.pad(src, (0, EPAD - E))
    dst = jnp.pad(dst, (0, EPAD - E), constant_values=NPAD - 1)
    zer = jnp.zeros((ROWS_PER_SUB, D), jnp.float32)

    p, q = _node_projections(x_pad, gate_w, gate_b)
    partials = _edge_kernel(x_pad, src, dst, p, q, zer)
    out = _combine(partials)
    return out.reshape(1, N, D)


# speedup vs baseline: 5.7332x; 1.2938x over previous
import functools

import jax
import jax.numpy as jnp
from jax import lax
from jax.experimental import pallas as pl
from jax.experimental.pallas import tpu as pltpu
from jax.experimental.pallas import tpu_sc as plsc

N = 10000
D = 128
E = 320000

NPAD = 10240
NW = 32
EPAD = 327680
PER_W = EPAD // NW
CHUNK = 64
N_CHUNKS = PER_W // CHUNK
ROWS_PER_SUB = NPAD // 16


def _pq_body(x_ref, w_ref, b_ref, p_ref, q_ref):
    xb = x_ref[...]
    w = w_ref[...]
    wa = w[0, :D]
    wb = w[0, D:]
    p_ref[...] = jnp.sum(xb * wa[None, :], axis=1)
    q_ref[...] = jnp.sum(xb * wb[None, :], axis=1) + b_ref[0]


def _node_projections(x_pad, gate_w, gate_b):
    return pl.pallas_call(
        _pq_body,
        out_shape=(
            jax.ShapeDtypeStruct((NPAD,), jnp.float32),
            jax.ShapeDtypeStruct((NPAD,), jnp.float32),
        ),
        in_specs=[
            pl.BlockSpec((NPAD, D), lambda: (0, 0)),
            pl.BlockSpec((1, 2 * D), lambda: (0, 0)),
            pl.BlockSpec(memory_space=pltpu.SMEM),
        ],
    )(x_pad, gate_w, gate_b)


def _lane_splat(vec, j):
    idx = jnp.full((16, 1), j, jnp.int32)
    dn = lax.GatherDimensionNumbers(
        offset_dims=(), collapsed_slice_dims=(0,), start_index_map=(0,))
    return lax.gather(vec, idx, dn, slice_sizes=(1,),
                      mode=lax.GatherScatterMode.PROMISE_IN_BOUNDS)


NBUF = 2
T_OUTER = N_CHUNKS // NBUF


def _edge_body(x_hbm, src_hbm, dst_hbm, p_hbm, q_hbm, zer_hbm, out_hbm,
               p_v, q_v, src_v, dst_v, att_v, xs_v, out_sh, sem_g, sem_s):
    if True:
        c = lax.axis_index("c")
        s = lax.axis_index("s")
        w = s * 2 + c

        pltpu.sync_copy(zer_hbm,
                        out_sh.at[pl.ds(s * ROWS_PER_SUB, ROWS_PER_SUB)])
        pltpu.sync_copy(p_hbm, p_v)
        pltpu.sync_copy(q_hbm, q_v)
        plsc.subcore_barrier()

        base = w * PER_W

        def load_idx(k, b):
            off = base + k * CHUNK
            pltpu.sync_copy(src_hbm.at[pl.ds(off, CHUNK)], src_v.at[b])
            pltpu.sync_copy(dst_hbm.at[pl.ds(off, CHUNK)], dst_v.at[b])

        def issue_gather(b):
            pltpu.async_copy(x_hbm.at[src_v.at[b]], xs_v.at[b], sem_g.at[b])

        def wait_bytes(sem, nrows):
            pltpu.make_async_copy(
                zer_hbm.at[pl.ds(0, nrows)], xs_v.at[0], sem).wait()

        def scores(b):
            for g in range(CHUNK // 16):
                sv = src_v[b, pl.ds(g * 16, 16)]
                dv = dst_v[b, pl.ds(g * 16, 16)]
                t = plsc.load_gather(p_v, [sv]) + plsc.load_gather(q_v, [dv])
                att_v[pl.ds(g * 16, 16)] = 1.0 / (1.0 + jnp.exp(-t))

        def scale(b):
            def scale_group(g, carry2):
                ag = att_v[pl.ds(g * 16, 16)]
                for j in range(16):
                    sj = _lane_splat(ag, j)
                    row = g * 16 + j
                    for dcol in range(D // 16):
                        sl = (b, row, pl.ds(dcol * 16, 16))
                        xs_v[sl] = xs_v[sl] * sj
                return carry2
            lax.fori_loop(0, CHUNK // 16, scale_group, 0)

        load_idx(0, 0)
        issue_gather(0)

        def outer_body(t, carry):
            for b in range(NBUF):
                k = t * NBUF + b
                nb = (b + 1) % NBUF
                scores(b)
                if b < NBUF - 1:
                    load_idx(k + 1, nb)
                    issue_gather(nb)
                else:
                    @pl.when(t < T_OUTER - 1)
                    def _():
                        load_idx(k + 1, nb)
                        issue_gather(nb)
                wait_bytes(sem_g.at[b], CHUNK)
                scale(b)
                pltpu.sync_copy(xs_v.at[b], out_sh.at[dst_v.at[b]], add=True)
            return carry

        lax.fori_loop(0, T_OUTER, outer_body, 0)
        plsc.subcore_barrier()
        r0 = s * ROWS_PER_SUB
        pltpu.sync_copy(out_sh.at[pl.ds(r0, ROWS_PER_SUB)],
                        out_hbm.at[c, pl.ds(r0, ROWS_PER_SUB)])


_edge_kernel = functools.partial(
    pl.kernel,
    out_type=jax.ShapeDtypeStruct((2, NPAD, D), jnp.float32),
    mesh=plsc.VectorSubcoreMesh(core_axis_name="c", subcore_axis_name="s"),
    compiler_params=pltpu.CompilerParams(needs_layout_passes=False),
    scratch_types=[
        pltpu.VMEM((NPAD,), jnp.float32),
        pltpu.VMEM((NPAD,), jnp.float32),
        pltpu.VMEM((NBUF, CHUNK), jnp.int32),
        pltpu.VMEM((NBUF, CHUNK), jnp.int32),
        pltpu.VMEM((CHUNK,), jnp.float32),
        pltpu.VMEM((NBUF, CHUNK, D), jnp.float32),
        pltpu.VMEM_SHARED((NPAD, D), jnp.float32),
        pltpu.SemaphoreType.DMA((NBUF,)),
        pltpu.SemaphoreType.DMA((NBUF,)),
    ],
)(_edge_body)


def _combine_body(part_ref, o_ref):
    o_ref[...] = part_ref[0] + part_ref[1]


def _combine(partials):
    return pl.pallas_call(
        _combine_body,
        grid=(5,),
        in_specs=[pl.BlockSpec((2, 2000, D), lambda i: (0, i, 0))],
        out_specs=pl.BlockSpec((2000, D), lambda i: (i, 0)),
        out_shape=jax.ShapeDtypeStruct((N, D), jnp.float32),
    )(partials)


def kernel(x, edge_index, gate_w, gate_b):
    x2d = x.reshape(N, D)
    x_pad = jnp.pad(x2d, ((0, NPAD - N), (0, 0)))
    src = edge_index[0].astype(jnp.int32)
    dst = edge_index[1].astype(jnp.int32)
    src = jnp.pad(src, (0, EPAD - E))
    dst = jnp.pad(dst, (0, EPAD - E), constant_values=NPAD - 1)
    zer = jnp.zeros((ROWS_PER_SUB, D), jnp.float32)

    p, q = _node_projections(x_pad, gate_w, gate_b)
    partials = _edge_kernel(x_pad, src, dst, p, q, zer)
    out = _combine(partials)
    return out.reshape(1, N, D)
